# baseline (speedup 1.0000x reference)
import jax
import jax.numpy as jnp
import numpy as np
from jax.experimental import pallas as pl
from jax.experimental.pallas import tpu as pltpu

_B = 64


def _sel_mat(w_in, kw, wp_pad):
    wp = (w_in - kw + 1) // 2
    xs = np.arange(w_in)[:, None, None]
    kj = np.arange(kw)[None, :, None]
    px = np.arange(wp_pad)[None, None, :]
    d = xs - 2 * px - kj
    sel = ((d == 0) | (d == 1)) & (px < wp)
    return sel.astype(np.float32)


def _row_taps(kh, taps):
    c = np.zeros((len(taps), kh), np.float32)
    for t, r in enumerate(taps):
        if 0 <= r <= kh:
            if r < kh:
                c[t, r] = 1.0
            if r >= 1:
                c[t, r - 1] = 1.0
    return c


def _lenet_block(x0_ref, x1_ref, x2_ref, u1_ref, u2_ref, w3_ref, w4_ref,
                 o_ref):
    b = o_ref.shape[1]
    xc = (x0_ref[:, 0], x1_ref[:, 0], x2_ref[:, 0])
    xv = jnp.stack(
        [jnp.concatenate([xc[c][:, m + j2, :] for c in range(3)
                          for j2 in range(2)], axis=1) for m in range(15)],
        axis=0).reshape(15 * b, 1536)

    a1e = jnp.dot(xv, u1_ref[0], preferred_element_type=jnp.float32)
    a1o = jnp.dot(xv, u1_ref[1], preferred_element_type=jnp.float32)
    p1e = jnp.maximum(a1e * 0.25, 0.0).astype(jnp.bfloat16).reshape(15, b, 512)
    p1o = jnp.maximum(a1o * 0.25, 0.0).astype(jnp.bfloat16).reshape(15, b, 512)

    ph = (p1e, p1o)
    fcat = jnp.concatenate(
        [ph[r % 2][r // 2:r // 2 + 13] for r in range(6)], axis=2)
    a2 = jnp.dot(fcat.reshape(13 * b, 3072), u2_ref[...],
                 preferred_element_type=jnp.float32)
    f = jnp.maximum(a2 * 0.25, 0.0).astype(jnp.bfloat16).reshape(13, b, 896)

    fl = jnp.concatenate([f[py] for py in range(13)], axis=1)
    h = jnp.dot(fl, w3_ref[...], preferred_element_type=jnp.float32)
    hb = jnp.maximum(h, 0.0).astype(jnp.bfloat16)

    o_ref[0] = jnp.dot(hb, w4_ref[...], preferred_element_type=jnp.float32)


def kernel(conv1_w, conv2_w, linear1_w, linear2_w, x):
    n, cin, hh, ww = x.shape
    c2, c1, kh, kw = conv2_w.shape
    hid, num_classes = linear1_w.shape[0], linear2_w.shape[0]
    wp1 = (ww - kw + 1) // 2
    wp2 = (wp1 - kw + 1) // 2
    hp2 = wp2
    wp1p, wp2p = 32, 14
    nb = n // _B

    c1q = np.zeros((2, 8, kh), np.float32)
    for s in range(2):
        c1q[s, 2 * s:2 * s + kh + 1] = _row_taps(kh, list(range(kh + 1)))
    sel1 = _sel_mat(ww, kw, wp1p)
    k1c = jnp.asarray((c1q.reshape(16, 1, kh, 1, 1)
                       * sel1[None, :, None, :, :]), jnp.bfloat16)
    u1 = jnp.einsum("qxyjp,oiyj->qixop", k1c, conv1_w.astype(jnp.bfloat16))
    u1 = u1.reshape(2, 2, 4, cin, ww, c1 * wp1p).transpose(0, 3, 1, 2, 4, 5)
    u1 = u1.reshape(2, 2 * cin * 4 * ww, c1 * wp1p)

    c2q = _row_taps(kh, list(range(kh + 1)))
    sel2 = np.pad(_sel_mat(wp1, kw, wp2p), ((0, wp1p - wp1), (0, 0), (0, 0)))
    k2c = jnp.asarray((c2q.reshape(kh + 1, 1, kh, 1, 1)
                       * sel2[None, :, None, :, :]), jnp.bfloat16)
    u2 = jnp.einsum("rxyjp,oiyj->rixop", k2c, conv2_w.astype(jnp.bfloat16))
    u2 = u2.reshape((kh + 1) * c1 * wp1p, c2 * wp2p)

    pad2 = jnp.asarray(np.eye(wp2, wp2p), jnp.bfloat16)
    w3b = jnp.einsum("hcyx,xp->ycph",
                     linear1_w.astype(jnp.bfloat16).reshape(hid, c2, hp2, wp2),
                     pad2)
    w3b = w3b.reshape(hp2 * c2 * wp2p, hid)
    w4b = linear2_w.T.astype(jnp.bfloat16)

    xs = x.astype(jnp.bfloat16).reshape(n, cin, hh // 4, 4 * ww)

    def _xmap(c):
        return lambda i: (i, c, 0, 0)

    x_specs = [pl.BlockSpec((_B, 1, hh // 4, 4 * ww), _xmap(c))
               for c in range(cin)]

    out = pl.pallas_call(
        _lenet_block,
        out_shape=jax.ShapeDtypeStruct((nb, _B, num_classes), jnp.float32),
        grid=(nb,),
        in_specs=x_specs + [
            pl.BlockSpec((2, 2 * cin * 4 * ww, c1 * wp1p), lambda i: (0, 0, 0)),
            pl.BlockSpec(((kh + 1) * c1 * wp1p, c2 * wp2p), lambda i: (0, 0)),
            pl.BlockSpec((hp2 * c2 * wp2p, hid), lambda i: (0, 0)),
            pl.BlockSpec((hid, num_classes), lambda i: (0, 0)),
        ],
        out_specs=pl.BlockSpec((1, _B, num_classes), lambda i: (i, 0, 0)),
        compiler_params=pltpu.CompilerParams(
            dimension_semantics=("arbitrary",)),
    )(xs, xs, xs, u1, u2, w3b, w4b)
    return out.reshape(n, num_classes)


# speedup vs baseline: 1.0959x; 1.0959x over previous
import jax
import jax.numpy as jnp
import numpy as np
from jax.experimental import pallas as pl
from jax.experimental.pallas import tpu as pltpu

_B = 64


def _sel_mat(w_in, kw, wp_pad):
    wp = (w_in - kw + 1) // 2
    xs = np.arange(w_in)[:, None, None]
    kj = np.arange(kw)[None, :, None]
    px = np.arange(wp_pad)[None, None, :]
    d = xs - 2 * px - kj
    sel = ((d == 0) | (d == 1)) & (px < wp)
    return sel.astype(np.float32)


def _row_taps(kh, taps):
    c = np.zeros((len(taps), kh), np.float32)
    for t, r in enumerate(taps):
        if 0 <= r <= kh:
            if r < kh:
                c[t, r] = 1.0
            if r >= 1:
                c[t, r - 1] = 1.0
    return c


def _lenet_block(x0_ref, x1_ref, x2_ref, u1_ref, u2_ref, w3_ref, w4_ref,
                 o_ref):
    b = o_ref.shape[1]
    xc = (x0_ref[:, 0].astype(jnp.bfloat16),
          x1_ref[:, 0].astype(jnp.bfloat16),
          x2_ref[:, 0].astype(jnp.bfloat16))
    xv = jnp.stack(
        [jnp.concatenate([xc[c][:, m + j2, :] for c in range(3)
                          for j2 in range(2)], axis=1) for m in range(15)],
        axis=0).reshape(15 * b, 1536)

    a1e = jnp.dot(xv, u1_ref[0], preferred_element_type=jnp.float32)
    a1o = jnp.dot(xv, u1_ref[1], preferred_element_type=jnp.float32)
    p1e = jnp.maximum(a1e * 0.25, 0.0).astype(jnp.bfloat16).reshape(15, b, 512)
    p1o = jnp.maximum(a1o * 0.25, 0.0).astype(jnp.bfloat16).reshape(15, b, 512)

    ph = (p1e, p1o)
    fcat = jnp.concatenate(
        [ph[r % 2][r // 2:r // 2 + 13] for r in range(6)], axis=2)
    a2 = jnp.dot(fcat.reshape(13 * b, 3072), u2_ref[...],
                 preferred_element_type=jnp.float32)
    f = jnp.maximum(a2 * 0.25, 0.0).astype(jnp.bfloat16).reshape(13, b, 896)

    fl = jnp.concatenate([f[py] for py in range(13)], axis=1)
    h = jnp.dot(fl, w3_ref[...], preferred_element_type=jnp.float32)
    hb = jnp.maximum(h, 0.0).astype(jnp.bfloat16)

    o_ref[0] = jnp.dot(hb, w4_ref[...], preferred_element_type=jnp.float32)


def kernel(conv1_w, conv2_w, linear1_w, linear2_w, x):
    n, cin, hh, ww = x.shape
    c2, c1, kh, kw = conv2_w.shape
    hid, num_classes = linear1_w.shape[0], linear2_w.shape[0]
    wp1 = (ww - kw + 1) // 2
    wp2 = (wp1 - kw + 1) // 2
    hp2 = wp2
    wp1p, wp2p = 32, 14
    nb = n // _B

    t1 = jnp.einsum("xjp,oiyj->yixop",
                    jnp.asarray(_sel_mat(ww, kw, wp1p), jnp.bfloat16),
                    conv1_w.astype(jnp.bfloat16))
    t1 = t1.reshape(kh, cin, ww, c1 * wp1p)
    z1 = jnp.zeros((1,) + t1.shape[1:], t1.dtype)
    taps1 = (jnp.concatenate([z1, t1, z1, z1], 0)
             + jnp.concatenate([t1, z1, z1, z1], 0))
    u1 = jnp.stack([taps1, jnp.roll(taps1, 2, axis=0)])
    u1 = u1.reshape(2, 2, 4, cin, ww, c1 * wp1p).transpose(0, 3, 1, 2, 4, 5)
    u1 = u1.reshape(2, 2 * cin * 4 * ww, c1 * wp1p)

    sel2 = np.pad(_sel_mat(wp1, kw, wp2p), ((0, wp1p - wp1), (0, 0), (0, 0)))
    t2 = jnp.einsum("xjp,oiyj->yixop", jnp.asarray(sel2, jnp.bfloat16),
                    conv2_w.astype(jnp.bfloat16))
    t2 = t2.reshape(kh, c1 * wp1p, c2 * wp2p)
    z2 = jnp.zeros((1,) + t2.shape[1:], t2.dtype)
    u2 = jnp.concatenate([z2, t2], 0) + jnp.concatenate([t2, z2], 0)
    u2 = u2.reshape((kh + 1) * c1 * wp1p, c2 * wp2p)

    pad2 = jnp.asarray(np.eye(wp2, wp2p), jnp.bfloat16)
    w3b = jnp.einsum("hcyx,xp->ycph",
                     linear1_w.astype(jnp.bfloat16).reshape(hid, c2, hp2, wp2),
                     pad2)
    w3b = w3b.reshape(hp2 * c2 * wp2p, hid)
    w4b = linear2_w.T.astype(jnp.bfloat16)

    xs = x.reshape(n, cin, hh // 4, 4 * ww)

    def _xmap(c):
        return lambda i: (i, c, 0, 0)

    x_specs = [pl.BlockSpec((_B, 1, hh // 4, 4 * ww), _xmap(c))
               for c in range(cin)]

    out = pl.pallas_call(
        _lenet_block,
        out_shape=jax.ShapeDtypeStruct((nb, _B, num_classes), jnp.float32),
        grid=(nb,),
        in_specs=x_specs + [
            pl.BlockSpec((2, 2 * cin * 4 * ww, c1 * wp1p), lambda i: (0, 0, 0)),
            pl.BlockSpec(((kh + 1) * c1 * wp1p, c2 * wp2p), lambda i: (0, 0)),
            pl.BlockSpec((hp2 * c2 * wp2p, hid), lambda i: (0, 0)),
            pl.BlockSpec((hid, num_classes), lambda i: (0, 0)),
        ],
        out_specs=pl.BlockSpec((1, _B, num_classes), lambda i: (i, 0, 0)),
        compiler_params=pltpu.CompilerParams(
            dimension_semantics=("arbitrary",)),
    )(xs, xs, xs, u1, u2, w3b, w4b)
    return out.reshape(n, num_classes)


# speedup vs baseline: 1.1767x; 1.0737x over previous
import jax
import jax.numpy as jnp
import numpy as np
from jax.experimental import pallas as pl
from jax.experimental.pallas import tpu as pltpu

_B = 64


def _sel_mat(w_in, kw, wp_pad):
    wp = (w_in - kw + 1) // 2
    xs = np.arange(w_in)[:, None, None]
    kj = np.arange(kw)[None, :, None]
    px = np.arange(wp_pad)[None, None, :]
    d = xs - 2 * px - kj
    sel = ((d == 0) | (d == 1)) & (px < wp)
    return sel.astype(np.float32)


def _row_taps(kh, taps):
    c = np.zeros((len(taps), kh), np.float32)
    for t, r in enumerate(taps):
        if 0 <= r <= kh:
            if r < kh:
                c[t, r] = 1.0
            if r >= 1:
                c[t, r - 1] = 1.0
    return c


def _lenet_block(x0_ref, x1_ref, x2_ref, u1_ref, u2_ref, w3_ref, w4_ref,
                 o_ref):
    b = o_ref.shape[1]
    xc = (x0_ref[:, 0].astype(jnp.bfloat16),
          x1_ref[:, 0].astype(jnp.bfloat16),
          x2_ref[:, 0].astype(jnp.bfloat16))
    xv = jnp.stack(
        [jnp.concatenate([xc[c][:, m + j2, :] for c in range(3)
                          for j2 in range(2)], axis=1) for m in range(15)],
        axis=0).reshape(15 * b, 1536)

    a1e = jnp.dot(xv, u1_ref[0], preferred_element_type=jnp.float32)
    a1o = jnp.dot(xv, u1_ref[1], preferred_element_type=jnp.float32)
    p1e = jnp.maximum(a1e * 0.25, 0.0).reshape(15, b, 512)
    p1o = jnp.maximum(a1o * 0.25, 0.0).reshape(15, b, 512)

    qe = (p1e + p1o).astype(jnp.bfloat16)
    qo = (p1o[0:14] + p1e[1:15]).astype(jnp.bfloat16)

    ph = (qe, qo)
    fcat = jnp.concatenate(
        [ph[k % 2][k // 2:k // 2 + 13] for k in range(5)], axis=2)
    a2 = jnp.dot(fcat.reshape(13 * b, 2560), u2_ref[...],
                 preferred_element_type=jnp.float32)
    f = jnp.maximum(a2 * 0.25, 0.0).astype(jnp.bfloat16).reshape(13, b, 896)

    fl = jnp.concatenate([f[py] for py in range(13)], axis=1)
    h = jnp.dot(fl, w3_ref[...], preferred_element_type=jnp.float32)
    hb = jnp.maximum(h, 0.0).astype(jnp.bfloat16)

    o_ref[0] = jnp.dot(hb, w4_ref[...], preferred_element_type=jnp.float32)


def kernel(conv1_w, conv2_w, linear1_w, linear2_w, x):
    n, cin, hh, ww = x.shape
    c2, c1, kh, kw = conv2_w.shape
    hid, num_classes = linear1_w.shape[0], linear2_w.shape[0]
    wp1 = (ww - kw + 1) // 2
    wp2 = (wp1 - kw + 1) // 2
    hp2 = wp2
    wp1p, wp2p = 32, 14
    nb = n // _B

    t1 = jnp.einsum("xjp,oiyj->yixop",
                    jnp.asarray(_sel_mat(ww, kw, wp1p), jnp.bfloat16),
                    conv1_w.astype(jnp.bfloat16))
    t1 = t1.reshape(kh, cin, ww, c1 * wp1p)
    z1 = jnp.zeros((1,) + t1.shape[1:], t1.dtype)
    taps1 = (jnp.concatenate([z1, t1, z1, z1], 0)
             + jnp.concatenate([t1, z1, z1, z1], 0))
    u1 = jnp.stack([taps1, jnp.roll(taps1, 2, axis=0)])
    u1 = u1.reshape(2, 2, 4, cin, ww, c1 * wp1p).transpose(0, 3, 1, 2, 4, 5)
    u1 = u1.reshape(2, 2 * cin * 4 * ww, c1 * wp1p)

    sel2 = np.pad(_sel_mat(wp1, kw, wp2p), ((0, wp1p - wp1), (0, 0), (0, 0)))
    t2 = jnp.einsum("xjp,oiyj->yixop", jnp.asarray(sel2, jnp.bfloat16),
                    conv2_w.astype(jnp.bfloat16))
    u2 = t2.reshape(kh * c1 * wp1p, c2 * wp2p)

    pad2 = jnp.asarray(np.eye(wp2, wp2p), jnp.bfloat16)
    w3b = jnp.einsum("hcyx,xp->ycph",
                     linear1_w.astype(jnp.bfloat16).reshape(hid, c2, hp2, wp2),
                     pad2)
    w3b = w3b.reshape(hp2 * c2 * wp2p, hid)
    w4b = linear2_w.T.astype(jnp.bfloat16)

    xs = x.reshape(n, cin, hh // 4, 4 * ww)

    def _xmap(c):
        return lambda i: (i, c, 0, 0)

    x_specs = [pl.BlockSpec((_B, 1, hh // 4, 4 * ww), _xmap(c))
               for c in range(cin)]

    out = pl.pallas_call(
        _lenet_block,
        out_shape=jax.ShapeDtypeStruct((nb, _B, num_classes), jnp.float32),
        grid=(nb,),
        in_specs=x_specs + [
            pl.BlockSpec((2, 2 * cin * 4 * ww, c1 * wp1p), lambda i: (0, 0, 0)),
            pl.BlockSpec((kh * c1 * wp1p, c2 * wp2p), lambda i: (0, 0)),
            pl.BlockSpec((hp2 * c2 * wp2p, hid), lambda i: (0, 0)),
            pl.BlockSpec((hid, num_classes), lambda i: (0, 0)),
        ],
        out_specs=pl.BlockSpec((1, _B, num_classes), lambda i: (i, 0, 0)),
        compiler_params=pltpu.CompilerParams(
            dimension_semantics=("arbitrary",)),
    )(xs, xs, xs, u1, u2, w3b, w4b)
    return out.reshape(n, num_classes)
